# baseline (speedup 1.0000x reference)
import functools

import jax
import jax.numpy as jnp
from jax.experimental import pallas as pl
from jax.experimental.pallas import tpu as pltpu

_VMEM_LIMIT = 64 * 1024 * 1024


def _interp_matrix_t(n_in: int, n_out: int) -> jnp.ndarray:
    if n_out == 1 or n_in == 1:
        src = jnp.zeros((n_out,), dtype=jnp.float32)
    else:
        src = jnp.arange(n_out, dtype=jnp.float32) * ((n_in - 1) / (n_out - 1))
    i0 = jnp.clip(jnp.floor(src).astype(jnp.int32), 0, n_in - 1)
    i1 = jnp.clip(i0 + 1, 0, n_in - 1)
    frac = src - i0.astype(jnp.float32)
    m0 = jax.nn.one_hot(i0, n_in, dtype=jnp.float32) * (1.0 - frac)[:, None]
    m1 = jax.nn.one_hot(i1, n_in, dtype=jnp.float32) * frac[:, None]
    return (m0 + m1).T


def _up2x_kernel(x_ref, aht_ref, awt_ref, o_ref):
    _, bc, h, w = x_ref.shape
    h_out = aht_ref.shape[1]
    w_out = awt_ref.shape[1]

    xt = jnp.swapaxes(x_ref[0], 1, 2)
    v = jnp.dot(
        xt.reshape(bc * w, h), aht_ref[...],
        preferred_element_type=jnp.float32,
    ).reshape(bc, w, h_out)

    vt = jnp.swapaxes(v, 1, 2)
    out = jnp.dot(
        vt.reshape(bc * h_out, w), awt_ref[...],
        preferred_element_type=jnp.float32,
    )
    o_ref[...] = out.reshape(1, bc, h_out, w_out)


def kernel(x: jnp.ndarray) -> jnp.ndarray:
    n, c, h, w = x.shape
    h_out, w_out = 2 * h, 2 * w

    a_h_t = _interp_matrix_t(h, h_out)
    a_w_t = _interp_matrix_t(w, w_out)

    bc = c
    for cand_bc in (128, 64, 32, 16, 8, 4, 2, 1):
        if c % cand_bc == 0:
            bc = cand_bc
            break
    cb = c // bc
    num_blocks = n * cb

    flops = 2 * n * c * h * w * h_out + 2 * n * c * h_out * w * w_out
    bytes_accessed = n * c * (h * w + h_out * w_out) * 4

    out = pl.pallas_call(
        _up2x_kernel,
        out_shape=jax.ShapeDtypeStruct((n, c, h_out, w_out), x.dtype),
        grid_spec=pltpu.PrefetchScalarGridSpec(
            num_scalar_prefetch=0,
            grid=(num_blocks,),
            in_specs=[
                pl.BlockSpec((1, bc, h, w),
                             lambda i, cb=cb: (i // cb, i % cb, 0, 0)),
                pl.BlockSpec((h, h_out), lambda i: (0, 0)),
                pl.BlockSpec((w, w_out), lambda i: (0, 0)),
            ],
            out_specs=pl.BlockSpec((1, bc, h_out, w_out),
                                   lambda i, cb=cb: (i // cb, i % cb, 0, 0)),
        ),
        compiler_params=pltpu.CompilerParams(
            dimension_semantics=("parallel",),
            vmem_limit_bytes=_VMEM_LIMIT),
        cost_estimate=pl.CostEstimate(
            flops=int(flops), transcendentals=0,
            bytes_accessed=int(bytes_accessed)),
    )(x, a_h_t, a_w_t)

    return out
